# baseline (speedup 1.0000x reference)
import functools

import jax
import jax.numpy as jnp
from jax import lax
from jax.experimental import pallas as pl
from jax.experimental.pallas import tpu as pltpu
from jax.experimental.pallas import tpu_sc as plsc

N = 10000
E = 320000
D = 128
HID = 128
H = 8
HD = HID // H
SCALE = HD ** (-0.5)

NC = 2
NS = 16
HH = H // NC
HW = HID // NC
EPT = E // NS
C = 80
NCHUNK = EPT // C
GROUPS = C // 16
NPAD = 10240
SROWS = NPAD * HH // 128
MROWS = SROWS // NS * 2

_MESH = plsc.VectorSubcoreMesh(
    core_axis_name="c", subcore_axis_name="s", num_cores=NC, num_subcores=NS)

BLK = 1000


def _qkv_body(x_ref, g_ref, b_ref, wq_ref, bq_ref, wk_ref, bk_ref,
              wv_ref, bv_ref, q_ref, k_ref, v_ref):
  xb = x_ref[...]
  mu = jnp.mean(xb, axis=-1, keepdims=True)
  xc = xb - mu
  var = jnp.mean(xc * xc, axis=-1, keepdims=True)
  xn = xc * lax.rsqrt(var + 1e-5) * g_ref[...] + b_ref[...]
  q_ref[...] = jnp.dot(xn, wq_ref[0], preferred_element_type=jnp.float32) + bq_ref[0]
  k_ref[...] = jnp.dot(xn, wk_ref[0], preferred_element_type=jnp.float32) + bk_ref[0]
  v_ref[...] = jnp.dot(xn, wv_ref[0], preferred_element_type=jnp.float32) + bv_ref[0]


def _qkv(x, g2, b2, wqT, bq2, wkT, bk2, wvT, bv2):
  row = pl.BlockSpec((BLK, D), lambda c, i: (i, 0))
  one = pl.BlockSpec((1, D), lambda c, i: (0, 0))
  wsp = pl.BlockSpec((1, D, HW), lambda c, i: (c, 0, 0))
  bsp = pl.BlockSpec((1, 1, HW), lambda c, i: (c, 0, 0))
  out = jax.ShapeDtypeStruct((NC * N, HW), jnp.float32)
  osp = pl.BlockSpec((BLK, HW), lambda c, i: (c * (N // BLK) + i, 0))
  return pl.pallas_call(
      _qkv_body,
      grid=(NC, N // BLK),
      in_specs=[row, one, one, wsp, bsp, wsp, bsp, wsp, bsp],
      out_specs=[osp, osp, osp],
      out_shape=[out, out, out],
  )(x, g2, b2, wqT, bq2, wkT, bk2, wvT, bv2)


def _scores_body(q_hbm, k_hbm, ei_hbm, ew_hbm, ws_out, smax_out,
                 idxb, idx2, qbuf, kbuf, ewbuf, wsbig, smax,
                 mrg_in, mrg_acc, slots_sh, sem_i, sem_g):
  cid = lax.axis_index("c")
  sid = lax.axis_index("s")
  iota = lax.iota(jnp.int32, 16)
  neg = jnp.full((16,), -3.0e38, jnp.float32)
  cofs = jnp.broadcast_to((cid * N).astype(jnp.int32), (16,))
  zeros16 = jnp.zeros((16,), jnp.int32)
  ones16 = jnp.full((16,), 1, jnp.int32)
  BMAX = E - C

  def init_body(i, carry):
    for j in range(8):
      smax[i, pl.ds(j * 16, 16)] = neg
    return carry
  lax.fori_loop(0, SROWS, init_body, 0)

  def ebase(cc):
    return jnp.minimum(sid * EPT + cc * C, BMAX)

  def issue_idx(cc):
    s3 = lax.rem(cc, 3)
    pltpu.async_copy(ei_hbm.at[pl.ds(ebase(cc), C)], idxb.at[s3], sem_i)

  def wait_idx():
    pltpu.make_async_copy(ei_hbm.at[pl.ds(0, C)], idxb.at[0], sem_i).wait()

  def compute_idx2(cc):
    s3 = lax.rem(cc, 3)
    s2 = lax.rem(cc, 2)
    s3v = jnp.broadcast_to(s3.astype(jnp.int32), (16,))
    for g in range(GROUPS):
      rows = jnp.full((16,), g * 16, jnp.int32) + iota
      og = plsc.load_gather(idxb, [s3v, rows, zeros16])
      dg = plsc.load_gather(idxb, [s3v, rows, ones16])
      idx2[s2, 0, pl.ds(g * 16, 16)] = og + cofs
      idx2[s2, 1, pl.ds(g * 16, 16)] = dg + cofs

  def issue_gathers(cc):
    s2 = lax.rem(cc, 2)
    pltpu.async_copy(q_hbm.at[idx2.at[s2, 0]], qbuf.at[pl.ds(s2 * C, C)],
                     sem_g)
    pltpu.async_copy(k_hbm.at[idx2.at[s2, 1]], kbuf.at[pl.ds(s2 * C, C)],
                     sem_g)
    pltpu.async_copy(ew_hbm.at[pl.ds(ebase(cc), C)],
                     ewbuf.at[pl.ds(s2 * C, C)], sem_g)

  def wait_gathers(cc):
    s2 = lax.rem(cc, 2)
    pltpu.make_async_copy(q_hbm.at[idx2.at[s2, 0]],
                          qbuf.at[pl.ds(s2 * C, C)], sem_g).wait()
    pltpu.make_async_copy(k_hbm.at[idx2.at[s2, 1]],
                          kbuf.at[pl.ds(s2 * C, C)], sem_g).wait()
    pltpu.make_async_copy(ew_hbm.at[pl.ds(0, C)],
                          ewbuf.at[pl.ds(s2 * C, C)], sem_g).wait()

  issue_idx(jnp.int32(0))
  wait_idx()
  issue_idx(jnp.int32(1))
  compute_idx2(jnp.int32(0))
  issue_gathers(jnp.int32(0))

  def chunk_body(c, carry):
    wait_idx()
    issue_idx(c + 2)
    compute_idx2(c + 1)
    issue_gathers(c + 1)
    wait_gathers(c)

    s3 = lax.rem(c, 3)
    s2 = lax.rem(c, 2)
    s3v = jnp.broadcast_to(s3.astype(jnp.int32), (16,))
    sr = jnp.broadcast_to((s2 * C).astype(jnp.int32), (16,))
    woff = jnp.broadcast_to((lax.rem(c, 5) * (C * HH)).astype(jnp.int32),
                            (16,))
    for g in range(GROUPS):
      rows = jnp.full((16,), g * 16, jnp.int32) + iota
      r2 = sr + rows
      orig = plsc.load_gather(idxb, [s3v, rows, zeros16])
      orig4 = orig * HH
      for hl in range(HH):
        qv = [plsc.load_gather(qbuf, [r2, jnp.full((16,), hl * HD + d,
                                                   jnp.int32)])
              for d in range(HD)]
        kv = [plsc.load_gather(kbuf, [r2, jnp.full((16,), hl * HD + d,
                                                   jnp.int32)])
              for d in range(HD)]
        pr = [qv[d] * kv[d] for d in range(HD)]
        s4 = [pr[j] + pr[j + 4] + pr[j + 8] + pr[j + 12] for j in range(4)]
        acc = (s4[0] + s4[1]) + (s4[2] + s4[3])
        hg = jnp.broadcast_to((cid * HH + hl).astype(jnp.int32), (16,))
        ew_v = plsc.load_gather(ewbuf, [r2, hg])
        ws_v = acc * SCALE * ew_v
        plsc.store_scatter(
            wsbig, [woff + rows * HH + jnp.full((16,), hl, jnp.int32)], ws_v)
        flat = orig4 + hl
        frow = lax.shift_right_logical(flat, 7)
        fcol = lax.bitwise_and(flat, 127)
        cur = plsc.load_gather(smax, [frow, fcol])
        wr = ws_v > cur
        plsc.store_scatter(smax, [frow, fcol], ws_v, mask=wr)
        cur2 = plsc.load_gather(smax, [frow, fcol])
        pend = jnp.max(jnp.where(cur2 < ws_v, 1, 0).astype(jnp.int32))

        @pl.when(pend > 0)
        def _fixup(frow=frow, fcol=fcol, ws_v=ws_v):
          def rmw_body(i, carry):
            cc2 = plsc.load_gather(smax, [frow, fcol])
            w2 = ws_v > cc2
            plsc.store_scatter(smax, [frow, fcol], ws_v, mask=w2)
            return carry
          lax.fori_loop(0, 15, rmw_body, 0)

    @pl.when(lax.rem(c, 5) == 4)
    def _flush():
      off = cid * (E * HH) + (sid * EPT + (c - 4) * C) * HH
      pltpu.sync_copy(wsbig, ws_out.at[pl.ds(off, 5 * C * HH)])
    return carry
  lax.fori_loop(0, NCHUNK, chunk_body, 0)

  wait_idx()
  wait_gathers(jnp.int32(NCHUNK))

  pltpu.sync_copy(smax, slots_sh.at[sid])
  plsc.subcore_barrier()

  @pl.when(sid < 8)
  def _merge():
    ro = sid * MROWS
    pltpu.sync_copy(slots_sh.at[0, pl.ds(ro, MROWS)], mrg_acc)
    for s in range(1, NS):
      pltpu.sync_copy(slots_sh.at[s, pl.ds(ro, MROWS)], mrg_in)

      def mx_body(i, carry):
        for j in range(8):
          a = mrg_acc[i, pl.ds(j * 16, 16)]
          b = mrg_in[i, pl.ds(j * 16, 16)]
          mrg_acc[i, pl.ds(j * 16, 16)] = jnp.maximum(a, b)
        return carry
      lax.fori_loop(0, MROWS, mx_body, 0)
    pltpu.sync_copy(mrg_acc, smax_out.at[cid, pl.ds(ro, MROWS)])


_scores = functools.partial(
    pl.kernel,
    compiler_params=pltpu.CompilerParams(needs_layout_passes=False,
                                         use_tc_tiling_on_sc=False),
    out_type=(jax.ShapeDtypeStruct((NC * E * HH,), jnp.float32),
              jax.ShapeDtypeStruct((NC, SROWS, 128), jnp.float32)),
    mesh=_MESH,
    scratch_types=[
        pltpu.VMEM((3, C, 2), jnp.int32),
        pltpu.VMEM((2, 2, C), jnp.int32),
        pltpu.VMEM((2 * C, HW), jnp.float32),
        pltpu.VMEM((2 * C, HW), jnp.float32),
        pltpu.VMEM((2 * C, H), jnp.float32),
        pltpu.VMEM((5 * C * HH,), jnp.float32),
        pltpu.VMEM((SROWS, 128), jnp.float32),
        pltpu.VMEM((MROWS, 128), jnp.float32),
        pltpu.VMEM((MROWS, 128), jnp.float32),
        pltpu.VMEM_SHARED((NS, SROWS, 128), jnp.float32),
        pltpu.SemaphoreType.DMA,
        pltpu.SemaphoreType.DMA,
    ])(_scores_body)


def _agg_body(ws_hbm, ei_hbm, v_hbm, smax_hbm, vals_out, den_out,
              idxb, idx2, vbuf, wsbuf, exbuf, wvbuf, smaxc,
              vals_sh, den_sh, sem_i, sem_g, sem_s):
  cid = lax.axis_index("c")
  sid = lax.axis_index("s")
  iota = lax.iota(jnp.int32, 16)
  zero = jnp.zeros((16,), jnp.float32)
  cofs = jnp.broadcast_to((cid * N).astype(jnp.int32), (16,))
  zeros16 = jnp.zeros((16,), jnp.int32)
  ones16 = jnp.full((16,), 1, jnp.int32)
  BMAX = E - C

  pltpu.sync_copy(smax_hbm.at[cid], smaxc)

  def zwv_body(r, carry):
    for j in range(HW // 16):
      wvbuf[r, pl.ds(j * 16, 16)] = zero
    return carry
  lax.fori_loop(0, 2 * C, zwv_body, 0)

  def zex_body(i, carry):
    flat = i * 16 + iota
    plsc.store_scatter(exbuf, [flat // H, flat % H], zero)
    return carry
  lax.fori_loop(0, (2 * C * H) // 16, zex_body, 0)

  for j in range(NPAD // NS // C):
    o = sid * (NPAD // NS) + j * C
    pltpu.sync_copy(wvbuf.at[pl.ds(0, C)], vals_sh.at[pl.ds(o, C)])
    pltpu.sync_copy(exbuf.at[pl.ds(0, C)], den_sh.at[pl.ds(o, C)])
  plsc.subcore_barrier()

  def ebase(cc):
    return jnp.minimum(sid * EPT + cc * C, BMAX)

  def issue_idx(cc):
    s3 = lax.rem(cc, 3)
    pltpu.async_copy(ei_hbm.at[pl.ds(ebase(cc), C)], idxb.at[s3], sem_i)

  def wait_idx():
    pltpu.make_async_copy(ei_hbm.at[pl.ds(0, C)], idxb.at[0], sem_i).wait()

  def compute_idx2(cc):
    s3 = lax.rem(cc, 3)
    s3v = jnp.broadcast_to(s3.astype(jnp.int32), (16,))
    for g in range(GROUPS):
      rows = jnp.full((16,), g * 16, jnp.int32) + iota
      og = plsc.load_gather(idxb, [s3v, rows, zeros16])
      dg = plsc.load_gather(idxb, [s3v, rows, ones16])
      idx2[s3, 0, pl.ds(g * 16, 16)] = dg + cofs
      idx2[s3, 1, pl.ds(g * 16, 16)] = og

  def issue_gathers(cc):
    s3 = lax.rem(cc, 3)
    s2 = lax.rem(cc, 2)
    pltpu.async_copy(v_hbm.at[idx2.at[s3, 0]], vbuf.at[pl.ds(s2 * C, C)],
                     sem_g)
    woff = cid * (E * HH) + ebase(cc) * HH
    pltpu.async_copy(ws_hbm.at[pl.ds(woff, C * HH)],
                     wsbuf.at[pl.ds(s2 * (C * HH), C * HH)], sem_g)

  def wait_gathers(cc):
    s3 = lax.rem(cc, 3)
    s2 = lax.rem(cc, 2)
    pltpu.make_async_copy(v_hbm.at[idx2.at[s3, 0]],
                          vbuf.at[pl.ds(s2 * C, C)], sem_g).wait()
    pltpu.make_async_copy(ws_hbm.at[pl.ds(0, C * HH)],
                          wsbuf.at[pl.ds(s2 * (C * HH), C * HH)],
                          sem_g).wait()

  def issue_scatters(cc):
    s3 = lax.rem(cc, 3)
    s2 = lax.rem(cc, 2)
    pltpu.async_copy(exbuf.at[pl.ds(s2 * C, C)],
                     den_sh.at[idx2.at[s3, 1]], sem_s, add=True)
    pltpu.async_copy(wvbuf.at[pl.ds(s2 * C, C)],
                     vals_sh.at[idx2.at[s3, 1]], sem_s, add=True)

  def wait_scatters(cc):
    s3 = lax.rem(cc, 3)
    s2 = lax.rem(cc, 2)
    pltpu.make_async_copy(exbuf.at[pl.ds(s2 * C, C)],
                          den_sh.at[idx2.at[s3, 1]], sem_s).wait()
    pltpu.make_async_copy(wvbuf.at[pl.ds(s2 * C, C)],
                          vals_sh.at[idx2.at[s3, 1]], sem_s).wait()

  issue_idx(jnp.int32(0))
  wait_idx()
  issue_idx(jnp.int32(1))
  compute_idx2(jnp.int32(0))
  issue_gathers(jnp.int32(0))

  def chunk_body(c, carry):
    wait_idx()
    issue_idx(c + 2)
    compute_idx2(c + 1)
    issue_gathers(c + 1)
    wait_gathers(c)

    s3 = lax.rem(c, 3)
    s2 = lax.rem(c, 2)
    s3v = jnp.broadcast_to(s3.astype(jnp.int32), (16,))
    sr = jnp.broadcast_to((s2 * C).astype(jnp.int32), (16,))
    wsoff = jnp.broadcast_to((s2 * (C * HH)).astype(jnp.int32), (16,))
    for g in range(GROUPS):
      rows = jnp.full((16,), g * 16, jnp.int32) + iota
      r2 = sr + rows
      orig = plsc.load_gather(idxb, [s3v, rows, zeros16])
      orig4 = orig * HH
      for hl in range(HH):
        flat = orig4 + hl
        frow = lax.shift_right_logical(flat, 7)
        fcol = lax.bitwise_and(flat, 127)
        m_v = plsc.load_gather(smaxc, [frow, fcol])
        ws_v = plsc.load_gather(
            wsbuf, [wsoff + rows * HH + jnp.full((16,), hl, jnp.int32)])
        ex_v = jnp.exp(ws_v - m_v)
        hg = jnp.broadcast_to((cid * HH + hl).astype(jnp.int32), (16,))
        plsc.store_scatter(exbuf, [r2, hg], ex_v)
        vv = [plsc.load_gather(vbuf, [r2, jnp.full((16,), hl * HD + dd,
                                                   jnp.int32)])
              for dd in range(HD)]
        for dd in range(HD):
          col = jnp.full((16,), hl * HD + dd, jnp.int32)
          plsc.store_scatter(wvbuf, [r2, col], vv[dd] * ex_v)

    @pl.when(c > 0)
    def _drain():
      wait_scatters(c - 1)
    issue_scatters(c)
    return carry
  lax.fori_loop(0, NCHUNK, chunk_body, 0)

  wait_idx()
  wait_gathers(jnp.int32(NCHUNK))
  wait_scatters(jnp.int32(NCHUNK - 1))
  plsc.subcore_barrier()
  o = sid * (NPAD // NS)
  pltpu.sync_copy(vals_sh.at[pl.ds(o, NPAD // NS)],
                  vals_out.at[cid, pl.ds(o, NPAD // NS)])
  pltpu.sync_copy(den_sh.at[pl.ds(o, NPAD // NS)],
                  den_out.at[cid, pl.ds(o, NPAD // NS)])


_agg = functools.partial(
    pl.kernel,
    compiler_params=pltpu.CompilerParams(needs_layout_passes=False,
                                         use_tc_tiling_on_sc=False),
    out_type=(jax.ShapeDtypeStruct((NC, NPAD, HW), jnp.float32),
              jax.ShapeDtypeStruct((NC, NPAD, H), jnp.float32)),
    mesh=_MESH,
    scratch_types=[
        pltpu.VMEM((3, C, 2), jnp.int32),
        pltpu.VMEM((3, 2, C), jnp.int32),
        pltpu.VMEM((2 * C, HW), jnp.float32),
        pltpu.VMEM((2 * C * HH,), jnp.float32),
        pltpu.VMEM((2 * C, H), jnp.float32),
        pltpu.VMEM((2 * C, HW), jnp.float32),
        pltpu.VMEM((SROWS, 128), jnp.float32),
        pltpu.VMEM_SHARED((NPAD, HW), jnp.float32),
        pltpu.VMEM_SHARED((NPAD, H), jnp.float32),
        pltpu.SemaphoreType.DMA,
        pltpu.SemaphoreType.DMA,
        pltpu.SemaphoreType.DMA,
    ])(_agg_body)


def _mlp_body(x_ref, vp_ref, dp_ref, w1_ref, b1_ref, w2_ref, b2_ref, o_ref):
  vals = jnp.concatenate([vp_ref[0], vp_ref[1]], axis=-1)
  den = dp_ref[0] + dp_ref[1]
  recip = 1.0 / (den + 1e-16)
  hh = lax.broadcasted_iota(jnp.int32, (H, HID), 0)
  cc = lax.broadcasted_iota(jnp.int32, (H, HID), 1) // HD
  rep = jnp.where(hh == cc, 1.0, 0.0).astype(jnp.float32)
  recip_e = jnp.dot(recip, rep, preferred_element_type=jnp.float32)
  vals = vals * recip_e
  y = jnp.dot(vals, w1_ref[...], preferred_element_type=jnp.float32) + b1_ref[...]
  z = jnp.dot(y, w2_ref[...], preferred_element_type=jnp.float32) + b2_ref[...]
  o_ref[...] = x_ref[...] + z


def _mlp(x, vals_p, den_p, w1T, b12, w2T, b22):
  return pl.pallas_call(
      _mlp_body,
      grid=(N // BLK,),
      in_specs=[
          pl.BlockSpec((BLK, D), lambda i: (i, 0)),
          pl.BlockSpec((NC, BLK, HW), lambda i: (0, i, 0)),
          pl.BlockSpec((NC, BLK, H), lambda i: (0, i, 0)),
          pl.BlockSpec((HID, HID), lambda i: (0, 0)),
          pl.BlockSpec((1, HID), lambda i: (0, 0)),
          pl.BlockSpec((HID, HID), lambda i: (0, 0)),
          pl.BlockSpec((1, HID), lambda i: (0, 0)),
      ],
      out_specs=pl.BlockSpec((BLK, D), lambda i: (i, 0)),
      out_shape=jax.ShapeDtypeStruct((N, D), jnp.float32),
  )(x, vals_p, den_p, w1T, b12, w2T, b22)


@jax.jit
def kernel(x, edge_index, edge_weights, ln_g, ln_b, Wq, bq, Wk, bk, Wv, bv,
           W1, b1, W2, b2):
  g2 = ln_g.reshape(1, D)
  lb2 = ln_b.reshape(1, D)
  def _split_w(w):
    wt = w.T
    return jnp.stack([wt[:, :HW], wt[:, HW:]])

  def _split_b(b):
    return b.reshape(NC, 1, HW)

  q, k, v = _qkv(x, g2, lb2, _split_w(Wq), _split_b(bq), _split_w(Wk),
                 _split_b(bk), _split_w(Wv), _split_b(bv))
  ei_t = edge_index.T
  ws, smax = _scores(q, k, ei_t, edge_weights)
  vals_p, den_p = _agg(ws, ei_t, v, smax)
  return _mlp(x, vals_p, den_p, W1.T, b1.reshape(1, HID), W2.T,
              b2.reshape(1, HID))


# speedup vs baseline: 1.1713x; 1.1713x over previous
import functools

import jax
import jax.numpy as jnp
from jax import lax
from jax.experimental import pallas as pl
from jax.experimental.pallas import tpu as pltpu
from jax.experimental.pallas import tpu_sc as plsc

N = 10000
E = 320000
D = 128
HID = 128
H = 8
HD = HID // H
SCALE = HD ** (-0.5)

NC = 2
NS = 16
HH = H // NC
HW = HID // NC
EPT = E // NS
C = 80
NCHUNK = EPT // C
GROUPS = C // 16
NPAD = 10240
SROWS = NPAD * HH // 128
MROWS = SROWS // NS * 2

_MESH = plsc.VectorSubcoreMesh(
    core_axis_name="c", subcore_axis_name="s", num_cores=NC, num_subcores=NS)

BLK = 1000


def _qkv_body(x_ref, g_ref, b_ref, wq_ref, bq_ref, wk_ref, bk_ref,
              wv_ref, bv_ref, q_ref, k_ref, v_ref):
  xb = x_ref[...]
  mu = jnp.mean(xb, axis=-1, keepdims=True)
  xc = xb - mu
  var = jnp.mean(xc * xc, axis=-1, keepdims=True)
  xn = xc * lax.rsqrt(var + 1e-5) * g_ref[...] + b_ref[...]
  q_ref[...] = jnp.dot(xn, wq_ref[0], preferred_element_type=jnp.float32) + bq_ref[0]
  k_ref[...] = jnp.dot(xn, wk_ref[0], preferred_element_type=jnp.float32) + bk_ref[0]
  v_ref[...] = jnp.dot(xn, wv_ref[0], preferred_element_type=jnp.float32) + bv_ref[0]


def _qkv(x, g2, b2, wqT, bq2, wkT, bk2, wvT, bv2):
  row = pl.BlockSpec((BLK, D), lambda c, i: (i, 0))
  one = pl.BlockSpec((1, D), lambda c, i: (0, 0))
  wsp = pl.BlockSpec((1, D, HW), lambda c, i: (c, 0, 0))
  bsp = pl.BlockSpec((1, 1, HW), lambda c, i: (c, 0, 0))
  out = jax.ShapeDtypeStruct((NC * N, HW), jnp.float32)
  osp = pl.BlockSpec((BLK, HW), lambda c, i: (c * (N // BLK) + i, 0))
  return pl.pallas_call(
      _qkv_body,
      grid=(NC, N // BLK),
      in_specs=[row, one, one, wsp, bsp, wsp, bsp, wsp, bsp],
      out_specs=[osp, osp, osp],
      out_shape=[out, out, out],
  )(x, g2, b2, wqT, bq2, wkT, bk2, wvT, bv2)


def _scores_body(q_hbm, k_hbm, ei_hbm, ew_hbm, ws_out, smax_out,
                 idxb, idx2, qbuf, kbuf, ewbuf, wsbig, smax,
                 mrg_in, mrg_acc, slots_sh, sem_i, sem_g):
  cid = lax.axis_index("c")
  sid = lax.axis_index("s")
  iota = lax.iota(jnp.int32, 16)
  neg = jnp.full((16,), -3.0e38, jnp.float32)
  cofs = jnp.broadcast_to((cid * N).astype(jnp.int32), (16,))
  zeros16 = jnp.zeros((16,), jnp.int32)
  ones16 = jnp.full((16,), 1, jnp.int32)
  BMAX = E - C

  def init_body(i, carry):
    for j in range(8):
      smax[i, pl.ds(j * 16, 16)] = neg
    return carry
  lax.fori_loop(0, SROWS, init_body, 0)

  def ebase(cc):
    return jnp.minimum(sid * EPT + cc * C, BMAX)

  def issue_idx(cc):
    s3 = lax.rem(cc, 3)
    pltpu.async_copy(ei_hbm.at[pl.ds(ebase(cc), C)], idxb.at[s3], sem_i)

  def wait_idx():
    pltpu.make_async_copy(ei_hbm.at[pl.ds(0, C)], idxb.at[0], sem_i).wait()

  def compute_idx2(cc):
    s3 = lax.rem(cc, 3)
    s2 = lax.rem(cc, 2)
    s3v = jnp.broadcast_to(s3.astype(jnp.int32), (16,))
    for g in range(GROUPS):
      rows = jnp.full((16,), g * 16, jnp.int32) + iota
      og = plsc.load_gather(idxb, [s3v, rows, zeros16])
      dg = plsc.load_gather(idxb, [s3v, rows, ones16])
      idx2[s2, 0, pl.ds(g * 16, 16)] = og + cofs
      idx2[s2, 1, pl.ds(g * 16, 16)] = dg + cofs

  def issue_gathers(cc):
    s2 = lax.rem(cc, 2)
    pltpu.async_copy(q_hbm.at[idx2.at[s2, 0]], qbuf.at[pl.ds(s2 * C, C)],
                     sem_g)
    pltpu.async_copy(k_hbm.at[idx2.at[s2, 1]], kbuf.at[pl.ds(s2 * C, C)],
                     sem_g)
    pltpu.async_copy(ew_hbm.at[pl.ds(ebase(cc), C)],
                     ewbuf.at[pl.ds(s2 * C, C)], sem_g)

  def wait_gathers(cc):
    s2 = lax.rem(cc, 2)
    pltpu.make_async_copy(q_hbm.at[idx2.at[s2, 0]],
                          qbuf.at[pl.ds(s2 * C, C)], sem_g).wait()
    pltpu.make_async_copy(k_hbm.at[idx2.at[s2, 1]],
                          kbuf.at[pl.ds(s2 * C, C)], sem_g).wait()
    pltpu.make_async_copy(ew_hbm.at[pl.ds(0, C)],
                          ewbuf.at[pl.ds(s2 * C, C)], sem_g).wait()

  issue_idx(jnp.int32(0))
  wait_idx()
  issue_idx(jnp.int32(1))
  compute_idx2(jnp.int32(0))
  issue_gathers(jnp.int32(0))

  def chunk_body(c, carry):
    wait_idx()
    issue_idx(c + 2)
    compute_idx2(c + 1)
    issue_gathers(c + 1)
    wait_gathers(c)

    s3 = lax.rem(c, 3)
    s2 = lax.rem(c, 2)
    s3v = jnp.broadcast_to(s3.astype(jnp.int32), (16,))
    sr = jnp.broadcast_to((s2 * C).astype(jnp.int32), (16,))
    woff = jnp.broadcast_to((lax.rem(c, 5) * (C * HH)).astype(jnp.int32),
                            (16,))
    for g in range(GROUPS):
      rows = jnp.full((16,), g * 16, jnp.int32) + iota
      r2 = sr + rows
      orig = plsc.load_gather(idxb, [s3v, rows, zeros16])
      orig4 = orig * HH
      for hl in range(HH):
        qv = [plsc.load_gather(qbuf, [r2, jnp.full((16,), hl * HD + d,
                                                   jnp.int32)])
              for d in range(HD)]
        kv = [plsc.load_gather(kbuf, [r2, jnp.full((16,), hl * HD + d,
                                                   jnp.int32)])
              for d in range(HD)]
        pr = [qv[d] * kv[d] for d in range(HD)]
        s4 = [pr[j] + pr[j + 4] + pr[j + 8] + pr[j + 12] for j in range(4)]
        acc = (s4[0] + s4[1]) + (s4[2] + s4[3])
        hg = jnp.broadcast_to((cid * HH + hl).astype(jnp.int32), (16,))
        ew_v = plsc.load_gather(ewbuf, [r2, hg])
        ws_v = acc * SCALE * ew_v
        plsc.store_scatter(
            wsbig, [woff + rows * HH + jnp.full((16,), hl, jnp.int32)], ws_v)
        flat = orig4 + hl
        frow = lax.shift_right_logical(flat, 7)
        fcol = lax.bitwise_and(flat, 127)

        def rmw_cond(p):
          return jnp.max(p) > 0

        def rmw_body(p):
          pm = p > 0
          cur = plsc.load_gather(smax, [frow, fcol])
          wr = jnp.logical_and(pm, ws_v > cur)
          plsc.store_scatter(smax, [frow, fcol], ws_v, mask=wr)
          cur2 = plsc.load_gather(smax, [frow, fcol])
          still = jnp.logical_and(pm, cur2 < ws_v)
          return jnp.where(still, 1, 0).astype(jnp.int32)

        lax.while_loop(rmw_cond, rmw_body, jnp.ones((16,), jnp.int32))

    @pl.when(lax.rem(c, 5) == 4)
    def _flush():
      off = cid * (E * HH) + (sid * EPT + (c - 4) * C) * HH
      pltpu.sync_copy(wsbig, ws_out.at[pl.ds(off, 5 * C * HH)])
    return carry
  lax.fori_loop(0, NCHUNK, chunk_body, 0)

  wait_idx()
  wait_gathers(jnp.int32(NCHUNK))

  pltpu.sync_copy(smax, slots_sh.at[sid])
  plsc.subcore_barrier()

  @pl.when(sid < 8)
  def _merge():
    ro = sid * MROWS
    pltpu.sync_copy(slots_sh.at[0, pl.ds(ro, MROWS)], mrg_acc)
    for s in range(1, NS):
      pltpu.sync_copy(slots_sh.at[s, pl.ds(ro, MROWS)], mrg_in)

      def mx_body(i, carry):
        for j in range(8):
          a = mrg_acc[i, pl.ds(j * 16, 16)]
          b = mrg_in[i, pl.ds(j * 16, 16)]
          mrg_acc[i, pl.ds(j * 16, 16)] = jnp.maximum(a, b)
        return carry
      lax.fori_loop(0, MROWS, mx_body, 0)
    pltpu.sync_copy(mrg_acc, smax_out.at[cid, pl.ds(ro, MROWS)])


_scores = functools.partial(
    pl.kernel,
    compiler_params=pltpu.CompilerParams(needs_layout_passes=False,
                                         use_tc_tiling_on_sc=False),
    out_type=(jax.ShapeDtypeStruct((NC * E * HH,), jnp.float32),
              jax.ShapeDtypeStruct((NC, SROWS, 128), jnp.float32)),
    mesh=_MESH,
    scratch_types=[
        pltpu.VMEM((3, C, 2), jnp.int32),
        pltpu.VMEM((2, 2, C), jnp.int32),
        pltpu.VMEM((2 * C, HW), jnp.float32),
        pltpu.VMEM((2 * C, HW), jnp.float32),
        pltpu.VMEM((2 * C, H), jnp.float32),
        pltpu.VMEM((5 * C * HH,), jnp.float32),
        pltpu.VMEM((SROWS, 128), jnp.float32),
        pltpu.VMEM((MROWS, 128), jnp.float32),
        pltpu.VMEM((MROWS, 128), jnp.float32),
        pltpu.VMEM_SHARED((NS, SROWS, 128), jnp.float32),
        pltpu.SemaphoreType.DMA,
        pltpu.SemaphoreType.DMA,
    ])(_scores_body)


def _agg_body(ws_hbm, ei_hbm, v_hbm, smax_hbm, vals_out, den_out,
              idxb, idx2, vbuf, wsbuf, exbuf, wvbuf, smaxc,
              vals_sh, den_sh, sem_i, sem_g, sem_s):
  cid = lax.axis_index("c")
  sid = lax.axis_index("s")
  iota = lax.iota(jnp.int32, 16)
  zero = jnp.zeros((16,), jnp.float32)
  cofs = jnp.broadcast_to((cid * N).astype(jnp.int32), (16,))
  zeros16 = jnp.zeros((16,), jnp.int32)
  ones16 = jnp.full((16,), 1, jnp.int32)
  BMAX = E - C

  pltpu.sync_copy(smax_hbm.at[cid], smaxc)

  def zwv_body(r, carry):
    for j in range(HW // 16):
      wvbuf[r, pl.ds(j * 16, 16)] = zero
    return carry
  lax.fori_loop(0, 2 * C, zwv_body, 0)

  def zex_body(i, carry):
    flat = i * 16 + iota
    plsc.store_scatter(exbuf, [flat // H, flat % H], zero)
    return carry
  lax.fori_loop(0, (2 * C * H) // 16, zex_body, 0)

  for j in range(NPAD // NS // C):
    o = sid * (NPAD // NS) + j * C
    pltpu.sync_copy(wvbuf.at[pl.ds(0, C)], vals_sh.at[pl.ds(o, C)])
    pltpu.sync_copy(exbuf.at[pl.ds(0, C)], den_sh.at[pl.ds(o, C)])
  plsc.subcore_barrier()

  def ebase(cc):
    return jnp.minimum(sid * EPT + cc * C, BMAX)

  def issue_idx(cc):
    s3 = lax.rem(cc, 3)
    pltpu.async_copy(ei_hbm.at[pl.ds(ebase(cc), C)], idxb.at[s3], sem_i)

  def wait_idx():
    pltpu.make_async_copy(ei_hbm.at[pl.ds(0, C)], idxb.at[0], sem_i).wait()

  def compute_idx2(cc):
    s3 = lax.rem(cc, 3)
    s3v = jnp.broadcast_to(s3.astype(jnp.int32), (16,))
    for g in range(GROUPS):
      rows = jnp.full((16,), g * 16, jnp.int32) + iota
      og = plsc.load_gather(idxb, [s3v, rows, zeros16])
      dg = plsc.load_gather(idxb, [s3v, rows, ones16])
      idx2[s3, 0, pl.ds(g * 16, 16)] = dg + cofs
      idx2[s3, 1, pl.ds(g * 16, 16)] = og

  def issue_gathers(cc):
    s3 = lax.rem(cc, 3)
    s2 = lax.rem(cc, 2)
    pltpu.async_copy(v_hbm.at[idx2.at[s3, 0]], vbuf.at[pl.ds(s2 * C, C)],
                     sem_g)
    woff = cid * (E * HH) + ebase(cc) * HH
    pltpu.async_copy(ws_hbm.at[pl.ds(woff, C * HH)],
                     wsbuf.at[pl.ds(s2 * (C * HH), C * HH)], sem_g)

  def wait_gathers(cc):
    s3 = lax.rem(cc, 3)
    s2 = lax.rem(cc, 2)
    pltpu.make_async_copy(v_hbm.at[idx2.at[s3, 0]],
                          vbuf.at[pl.ds(s2 * C, C)], sem_g).wait()
    pltpu.make_async_copy(ws_hbm.at[pl.ds(0, C * HH)],
                          wsbuf.at[pl.ds(s2 * (C * HH), C * HH)],
                          sem_g).wait()

  def issue_scatters(cc):
    s3 = lax.rem(cc, 3)
    s2 = lax.rem(cc, 2)
    pltpu.async_copy(exbuf.at[pl.ds(s2 * C, C)],
                     den_sh.at[idx2.at[s3, 1]], sem_s, add=True)
    pltpu.async_copy(wvbuf.at[pl.ds(s2 * C, C)],
                     vals_sh.at[idx2.at[s3, 1]], sem_s, add=True)

  def wait_scatters(cc):
    s3 = lax.rem(cc, 3)
    s2 = lax.rem(cc, 2)
    pltpu.make_async_copy(exbuf.at[pl.ds(s2 * C, C)],
                          den_sh.at[idx2.at[s3, 1]], sem_s).wait()
    pltpu.make_async_copy(wvbuf.at[pl.ds(s2 * C, C)],
                          vals_sh.at[idx2.at[s3, 1]], sem_s).wait()

  issue_idx(jnp.int32(0))
  wait_idx()
  issue_idx(jnp.int32(1))
  compute_idx2(jnp.int32(0))
  issue_gathers(jnp.int32(0))

  def chunk_body(c, carry):
    wait_idx()
    issue_idx(c + 2)
    compute_idx2(c + 1)
    issue_gathers(c + 1)
    wait_gathers(c)

    s3 = lax.rem(c, 3)
    s2 = lax.rem(c, 2)
    s3v = jnp.broadcast_to(s3.astype(jnp.int32), (16,))
    sr = jnp.broadcast_to((s2 * C).astype(jnp.int32), (16,))
    wsoff = jnp.broadcast_to((s2 * (C * HH)).astype(jnp.int32), (16,))
    for g in range(GROUPS):
      rows = jnp.full((16,), g * 16, jnp.int32) + iota
      r2 = sr + rows
      orig = plsc.load_gather(idxb, [s3v, rows, zeros16])
      orig4 = orig * HH
      for hl in range(HH):
        flat = orig4 + hl
        frow = lax.shift_right_logical(flat, 7)
        fcol = lax.bitwise_and(flat, 127)
        m_v = plsc.load_gather(smaxc, [frow, fcol])
        ws_v = plsc.load_gather(
            wsbuf, [wsoff + rows * HH + jnp.full((16,), hl, jnp.int32)])
        ex_v = jnp.exp(ws_v - m_v)
        hg = jnp.broadcast_to((cid * HH + hl).astype(jnp.int32), (16,))
        plsc.store_scatter(exbuf, [r2, hg], ex_v)
        vv = [plsc.load_gather(vbuf, [r2, jnp.full((16,), hl * HD + dd,
                                                   jnp.int32)])
              for dd in range(HD)]
        for dd in range(HD):
          col = jnp.full((16,), hl * HD + dd, jnp.int32)
          plsc.store_scatter(wvbuf, [r2, col], vv[dd] * ex_v)

    @pl.when(c > 0)
    def _drain():
      wait_scatters(c - 1)
    issue_scatters(c)
    return carry
  lax.fori_loop(0, NCHUNK, chunk_body, 0)

  wait_idx()
  wait_gathers(jnp.int32(NCHUNK))
  wait_scatters(jnp.int32(NCHUNK - 1))
  plsc.subcore_barrier()
  o = sid * (NPAD // NS)
  pltpu.sync_copy(vals_sh.at[pl.ds(o, NPAD // NS)],
                  vals_out.at[cid, pl.ds(o, NPAD // NS)])
  pltpu.sync_copy(den_sh.at[pl.ds(o, NPAD // NS)],
                  den_out.at[cid, pl.ds(o, NPAD // NS)])


_agg = functools.partial(
    pl.kernel,
    compiler_params=pltpu.CompilerParams(needs_layout_passes=False,
                                         use_tc_tiling_on_sc=False),
    out_type=(jax.ShapeDtypeStruct((NC, NPAD, HW), jnp.float32),
              jax.ShapeDtypeStruct((NC, NPAD, H), jnp.float32)),
    mesh=_MESH,
    scratch_types=[
        pltpu.VMEM((3, C, 2), jnp.int32),
        pltpu.VMEM((3, 2, C), jnp.int32),
        pltpu.VMEM((2 * C, HW), jnp.float32),
        pltpu.VMEM((2 * C * HH,), jnp.float32),
        pltpu.VMEM((2 * C, H), jnp.float32),
        pltpu.VMEM((2 * C, HW), jnp.float32),
        pltpu.VMEM((SROWS, 128), jnp.float32),
        pltpu.VMEM_SHARED((NPAD, HW), jnp.float32),
        pltpu.VMEM_SHARED((NPAD, H), jnp.float32),
        pltpu.SemaphoreType.DMA,
        pltpu.SemaphoreType.DMA,
        pltpu.SemaphoreType.DMA,
    ])(_agg_body)


def _mlp_body(x_ref, vp_ref, dp_ref, w1_ref, b1_ref, w2_ref, b2_ref, o_ref):
  vals = jnp.concatenate([vp_ref[0], vp_ref[1]], axis=-1)
  den = dp_ref[0] + dp_ref[1]
  recip = 1.0 / (den + 1e-16)
  hh = lax.broadcasted_iota(jnp.int32, (H, HID), 0)
  cc = lax.broadcasted_iota(jnp.int32, (H, HID), 1) // HD
  rep = jnp.where(hh == cc, 1.0, 0.0).astype(jnp.float32)
  recip_e = jnp.dot(recip, rep, preferred_element_type=jnp.float32)
  vals = vals * recip_e
  y = jnp.dot(vals, w1_ref[...], preferred_element_type=jnp.float32) + b1_ref[...]
  z = jnp.dot(y, w2_ref[...], preferred_element_type=jnp.float32) + b2_ref[...]
  o_ref[...] = x_ref[...] + z


def _mlp(x, vals_p, den_p, w1T, b12, w2T, b22):
  return pl.pallas_call(
      _mlp_body,
      grid=(N // BLK,),
      in_specs=[
          pl.BlockSpec((BLK, D), lambda i: (i, 0)),
          pl.BlockSpec((NC, BLK, HW), lambda i: (0, i, 0)),
          pl.BlockSpec((NC, BLK, H), lambda i: (0, i, 0)),
          pl.BlockSpec((HID, HID), lambda i: (0, 0)),
          pl.BlockSpec((1, HID), lambda i: (0, 0)),
          pl.BlockSpec((HID, HID), lambda i: (0, 0)),
          pl.BlockSpec((1, HID), lambda i: (0, 0)),
      ],
      out_specs=pl.BlockSpec((BLK, D), lambda i: (i, 0)),
      out_shape=jax.ShapeDtypeStruct((N, D), jnp.float32),
  )(x, vals_p, den_p, w1T, b12, w2T, b22)


@jax.jit
def kernel(x, edge_index, edge_weights, ln_g, ln_b, Wq, bq, Wk, bk, Wv, bv,
           W1, b1, W2, b2):
  g2 = ln_g.reshape(1, D)
  lb2 = ln_b.reshape(1, D)
  def _split_w(w):
    wt = w.T
    return jnp.stack([wt[:, :HW], wt[:, HW:]])

  def _split_b(b):
    return b.reshape(NC, 1, HW)

  q, k, v = _qkv(x, g2, lb2, _split_w(Wq), _split_b(bq), _split_w(Wk),
                 _split_b(bk), _split_w(Wv), _split_b(bv))
  ei_t = edge_index.T
  ws, smax = _scores(q, k, ei_t, edge_weights)
  vals_p, den_p = _agg(ws, ei_t, v, smax)
  return _mlp(x, vals_p, den_p, W1.T, b1.reshape(1, HID), W2.T,
              b2.reshape(1, HID))
